# Initial kernel scaffold; baseline (speedup 1.0000x reference)
import jax
import jax.numpy as jnp
from jax.experimental import pallas as pl


def kernel(x, edge_index, query_edges, W1, b1, W2, b2, Wfc, bfc, Wd1, bd1, Wd2, bd2, Wd3, bd3):
    raise NotImplementedError("write your pallas kernel here")


# speedup vs baseline: 12.4794x; 12.4794x over previous
import functools

import jax
import jax.numpy as jnp
from jax import lax
from jax.experimental import pallas as pl
from jax.experimental.pallas import tpu as pltpu
from jax.experimental.pallas import tpu_sc as plsc

N_NODES = 10000
N_PAD = 10240
N_EDGES = 320000
N_QUERY = 100000
D = 128

NC = 2
NS = 16
NW = NC * NS

CE = 256
JE = CE // 128
NCH_E = N_EDGES // CE
ROWS_PER_TILE = N_PAD // NS

DEG_PER_TILE = N_PAD // NS

CQ = 256
Q_PAD = 102400
NCH_Q = Q_PAD // CQ


def _wid():
    return lax.axis_index("s") * NC + lax.axis_index("c")


def _sc_degree_body(dst2d_hbm, ones_hbm, out_hbm, idxv, onesv, zv, acc_sh):
    c = lax.axis_index("c")
    s = lax.axis_index("s")
    wid = _wid()
    for k in range(0, DEG_PER_TILE, 16):
        zv[pl.ds(k, 16)] = jnp.zeros((16,), jnp.float32)
    pltpu.sync_copy(zv, acc_sh.at[pl.ds(s * DEG_PER_TILE, DEG_PER_TILE)])
    pltpu.sync_copy(ones_hbm, onesv)
    plsc.subcore_barrier()

    n_rows = N_EDGES // 128
    my_n = (n_rows - wid + NW - 1) // NW

    def body(i, carry):
        r = wid + i * NW
        pltpu.sync_copy(dst2d_hbm.at[r], idxv)
        pltpu.sync_copy(onesv, acc_sh.at[idxv], add=True)
        return carry

    lax.fori_loop(0, my_n, body, 0)
    plsc.subcore_barrier()
    pltpu.sync_copy(
        acc_sh.at[pl.ds(s * DEG_PER_TILE, DEG_PER_TILE)],
        out_hbm.at[c, pl.ds(s * DEG_PER_TILE, DEG_PER_TILE)],
    )


def _sc_segsum_body(y_hbm, src2d_hbm, dst2d_hbm, z_hbm, out_hbm,
                    idxs, idxd, rows, acc_sh, sem):
    c = lax.axis_index("c")
    s = lax.axis_index("s")
    wid = _wid()
    row0 = s * ROWS_PER_TILE
    pltpu.sync_copy(z_hbm.at[pl.ds(row0, ROWS_PER_TILE)],
                    acc_sh.at[pl.ds(row0, ROWS_PER_TILE)])
    plsc.subcore_barrier()

    my_n = (NCH_E - wid + NW - 1) // NW

    def body(i, carry):
        k = wid + i * NW
        rb = k * JE
        pltpu.sync_copy(src2d_hbm.at[pl.ds(rb, JE)], idxs)
        pltpu.sync_copy(dst2d_hbm.at[pl.ds(rb, JE)], idxd)
        cps = [
            pltpu.async_copy(y_hbm.at[idxs.at[j]],
                             rows.at[pl.ds(j * 128, 128)], sem)
            for j in range(JE)
        ]
        for cp in cps:
            cp.wait()
        for j in range(JE):
            pltpu.sync_copy(rows.at[pl.ds(j * 128, 128)],
                            acc_sh.at[idxd.at[j]], add=True)
        return carry

    lax.fori_loop(0, my_n, body, 0)
    plsc.subcore_barrier()
    pltpu.sync_copy(acc_sh.at[pl.ds(row0, ROWS_PER_TILE)],
                    out_hbm.at[c, pl.ds(row0, ROWS_PER_TILE)])


def _sc_pairgather_body(u_hbm, v_hbm, qs2d_hbm, qd2d_hbm, gu_hbm, gv_hbm,
                        idxs, idxd, ru, rv, sem):
    wid = _wid()
    my_n = (NCH_Q - wid + NW - 1) // NW

    def body(i, carry):
        k = wid + i * NW
        rb = k * 2
        pltpu.sync_copy(qs2d_hbm.at[pl.ds(rb, 2)], idxs)
        pltpu.sync_copy(qd2d_hbm.at[pl.ds(rb, 2)], idxd)
        cps = []
        for j in range(2):
            cps.append(pltpu.async_copy(u_hbm.at[idxs.at[j]],
                                        ru.at[pl.ds(j * 128, 128)], sem))
            cps.append(pltpu.async_copy(v_hbm.at[idxd.at[j]],
                                        rv.at[pl.ds(j * 128, 128)], sem))
        for cp in cps:
            cp.wait()
        base = k * CQ
        pltpu.sync_copy(ru, gu_hbm.at[pl.ds(base, CQ)])
        pltpu.sync_copy(rv, gv_hbm.at[pl.ds(base, CQ)])
        return carry

    lax.fori_loop(0, my_n, body, 0)


@functools.cache
def _sc_kernels():
    mesh = plsc.VectorSubcoreMesh(core_axis_name="c", subcore_axis_name="s",
                                  num_cores=NC, num_subcores=NS)
    sc_degree = pl.kernel(
        _sc_degree_body,
        out_type=jax.ShapeDtypeStruct((NC, N_PAD), jnp.float32),
        mesh=mesh,
        scratch_types=[
            pltpu.VMEM((128,), jnp.int32),
            pltpu.VMEM((128,), jnp.float32),
            pltpu.VMEM((DEG_PER_TILE,), jnp.float32),
            pltpu.VMEM_SHARED((N_PAD,), jnp.float32),
        ],
    )
    sc_segsum = pl.kernel(
        _sc_segsum_body,
        out_type=jax.ShapeDtypeStruct((NC, N_PAD, D), jnp.float32),
        mesh=mesh,
        scratch_types=[
            pltpu.VMEM((JE, 128), jnp.int32),
            pltpu.VMEM((JE, 128), jnp.int32),
            pltpu.VMEM((CE, D), jnp.float32),
            pltpu.VMEM_SHARED((N_PAD, D), jnp.float32),
            pltpu.SemaphoreType.DMA,
        ],
    )
    sc_pairgather = pl.kernel(
        _sc_pairgather_body,
        out_type=[
            jax.ShapeDtypeStruct((Q_PAD, D), jnp.float32),
            jax.ShapeDtypeStruct((Q_PAD, D), jnp.float32),
        ],
        mesh=mesh,
        scratch_types=[
            pltpu.VMEM((2, 128), jnp.int32),
            pltpu.VMEM((2, 128), jnp.int32),
            pltpu.VMEM((CQ, D), jnp.float32),
            pltpu.VMEM((CQ, D), jnp.float32),
            pltpu.SemaphoreType.DMA,
        ],
    )
    return sc_degree, sc_segsum, sc_pairgather


_BN = 1024
_BQ = 2048


def _tc_prescale_body(x_ref, w1_ref, degp_ref, y1_ref, dinv_ref):
    deg = degp_ref[0, :] + degp_ref[1, :] + 1.0
    dinv = lax.rsqrt(deg)
    xw = jnp.dot(x_ref[...], w1_ref[...], preferred_element_type=jnp.float32)
    y1_ref[...] = xw * dinv[:, None]
    dinv_ref[...] = dinv


def _tc_mid_body(y1_ref, p0_ref, p1_ref, dinv_ref, b1_ref, w2_ref, y2_ref):
    dinv = dinv_ref[...]
    h = (p0_ref[...] + p1_ref[...] + y1_ref[...]) * dinv[:, None] + b1_ref[...][None, :]
    h = jnp.maximum(h, 0.0)
    y2_ref[...] = jnp.dot(h, w2_ref[...], preferred_element_type=jnp.float32) * dinv[:, None]


def _tc_final_body(y2_ref, q0_ref, q1_ref, dinv_ref, b2_ref, wfc_ref, bfc_ref,
                   wd1a_ref, wd1b_ref, bd1_ref, u_ref, v_ref):
    dinv = dinv_ref[...]
    h = (q0_ref[...] + q1_ref[...] + y2_ref[...]) * dinv[:, None] + b2_ref[...][None, :]
    h = jnp.maximum(h, 0.0)
    z = jnp.dot(h, wfc_ref[...], preferred_element_type=jnp.float32) + bfc_ref[...][None, :]
    u_ref[...] = jnp.dot(z, wd1a_ref[...], preferred_element_type=jnp.float32) + bd1_ref[...][None, :]
    v_ref[...] = jnp.dot(z, wd1b_ref[...], preferred_element_type=jnp.float32)


def _tc_dec_body(gu_ref, gv_ref, wd2_ref, bd2_ref, wd3_ref, bd3_ref, o_ref):
    t = jnp.maximum(gu_ref[...] + gv_ref[...], 0.0)
    t2 = jnp.dot(t, wd2_ref[...], preferred_element_type=jnp.float32) + bd2_ref[...][None, :]
    t2 = jnp.maximum(t2, 0.0)
    sc = jnp.sum(t2 * wd3_ref[...][None, :], axis=1) + bd3_ref[...]
    o_ref[...] = jax.nn.sigmoid(sc)


def _full(shape):
    return pl.BlockSpec(shape, lambda i: tuple(0 for _ in shape))


def kernel(x, edge_index, query_edges, W1, b1, W2, b2, Wfc, bfc,
           Wd1, bd1, Wd2, bd2, Wd3, bd3):
    src = edge_index[0].astype(jnp.int32)
    dst = edge_index[1].astype(jnp.int32)
    qs = query_edges[0].astype(jnp.int32)
    qd = query_edges[1].astype(jnp.int32)

    src2d = src.reshape(N_EDGES // 128, 128)
    dst2d = dst.reshape(N_EDGES // 128, 128)
    qpad = Q_PAD - N_QUERY
    qs2d = jnp.concatenate([qs, jnp.zeros((qpad,), jnp.int32)]).reshape(Q_PAD // 128, 128)
    qd2d = jnp.concatenate([qd, jnp.zeros((qpad,), jnp.int32)]).reshape(Q_PAD // 128, 128)
    ones_row = jnp.ones((128,), jnp.float32)
    xp = jnp.pad(x, ((0, N_PAD - N_NODES), (0, 0)))
    zeros2d = jnp.zeros((N_PAD, D), jnp.float32)

    _sc_degree, _sc_segsum, _sc_pairgather = _sc_kernels()

    degp = _sc_degree(dst2d, ones_row)

    grid_n = N_PAD // _BN
    y1, dinv = pl.pallas_call(
        _tc_prescale_body,
        grid=(grid_n,),
        in_specs=[
            pl.BlockSpec((_BN, D), lambda i: (i, 0)),
            _full((D, D)),
            pl.BlockSpec((NC, _BN), lambda i: (0, i)),
        ],
        out_specs=[
            pl.BlockSpec((_BN, D), lambda i: (i, 0)),
            pl.BlockSpec((_BN,), lambda i: (i,)),
        ],
        out_shape=[
            jax.ShapeDtypeStruct((N_PAD, D), jnp.float32),
            jax.ShapeDtypeStruct((N_PAD,), jnp.float32),
        ],
    )(xp, W1, degp)

    p = _sc_segsum(y1, src2d, dst2d, zeros2d)

    y2 = pl.pallas_call(
        _tc_mid_body,
        grid=(grid_n,),
        in_specs=[
            pl.BlockSpec((_BN, D), lambda i: (i, 0)),
            pl.BlockSpec((_BN, D), lambda i: (i, 0)),
            pl.BlockSpec((_BN, D), lambda i: (i, 0)),
            pl.BlockSpec((_BN,), lambda i: (i,)),
            _full((D,)),
            _full((D, D)),
        ],
        out_specs=pl.BlockSpec((_BN, D), lambda i: (i, 0)),
        out_shape=jax.ShapeDtypeStruct((N_PAD, D), jnp.float32),
    )(y1, p[0], p[1], dinv, b1, W2)

    q = _sc_segsum(y2, src2d, dst2d, zeros2d)

    Wd1a = Wd1[:D]
    Wd1b = Wd1[D:]
    u, v = pl.pallas_call(
        _tc_final_body,
        grid=(grid_n,),
        in_specs=[
            pl.BlockSpec((_BN, D), lambda i: (i, 0)),
            pl.BlockSpec((_BN, D), lambda i: (i, 0)),
            pl.BlockSpec((_BN, D), lambda i: (i, 0)),
            pl.BlockSpec((_BN,), lambda i: (i,)),
            _full((D,)),
            _full((D, D)),
            _full((D,)),
            _full((D, D)),
            _full((D, D)),
            _full((D,)),
        ],
        out_specs=[
            pl.BlockSpec((_BN, D), lambda i: (i, 0)),
            pl.BlockSpec((_BN, D), lambda i: (i, 0)),
        ],
        out_shape=[
            jax.ShapeDtypeStruct((N_PAD, D), jnp.float32),
            jax.ShapeDtypeStruct((N_PAD, D), jnp.float32),
        ],
    )(y2, q[0], q[1], dinv, b2, Wfc, bfc, Wd1a, Wd1b, bd1)

    gu, gv = _sc_pairgather(u, v, qs2d, qd2d)

    grid_q = Q_PAD // _BQ
    out = pl.pallas_call(
        _tc_dec_body,
        grid=(grid_q,),
        in_specs=[
            pl.BlockSpec((_BQ, D), lambda i: (i, 0)),
            pl.BlockSpec((_BQ, D), lambda i: (i, 0)),
            _full((D, D // 2)),
            _full((D // 2,)),
            _full((D // 2,)),
            _full((1,)),
        ],
        out_specs=pl.BlockSpec((_BQ,), lambda i: (i,)),
        out_shape=jax.ShapeDtypeStruct((Q_PAD,), jnp.float32),
    )(gu, gv, Wd2, bd2, Wd3[:, 0], bd3)

    return out[:N_QUERY]
